# baseline (speedup 1.0000x reference)
import jax
import jax.numpy as jnp
from jax import lax
from jax.experimental import pallas as pl
from jax.experimental.pallas import tpu as pltpu
from jax.experimental.pallas import tpu_sc as plsc

N = 10000
E = 320000
D = 128
F = 64
W = 72
NC = 2
NS = 16
NW = NC * NS
B = 128
NB = 79
CHUNK = NB * B
EP = NW * CHUNK
NRING = 6
ROWS = 10112
RPS = ROWS // NS
ZR = 79
RP = 632
RT = 1000
RM = 2000

T = 2 * NB
NG = T // NRING
TAIL = T - NG * NRING


def _sc_agg_body(tbl_hbm, idx_hbm, out0_hbm, out1_hbm, acc_s, src_v, dst_v,
                 gbuf0, gbuf1, gbuf2, gbuf3, gbuf4, gbuf5,
                 zbuf, gsems, ssems):
    c = lax.axis_index("c")
    s = lax.axis_index("s")
    w = c * NS + s

    idx_src_cp = pltpu.async_copy(idx_hbm.at[0, w], src_v, gsems.at[0])
    idx_dst_cp = pltpu.async_copy(idx_hbm.at[1, w], dst_v, gsems.at[1])

    z16 = jnp.zeros((16,), jnp.float32)

    def zero_row(r, _):
        for cc in range(W // 16):
            zbuf[r, pl.ds(cc * 16, 16)] = z16
        return _

    lax.fori_loop(0, ZR, zero_row, None)

    def zero_chunk(k, _):
        pltpu.sync_copy(zbuf, acc_s.at[pl.ds(s * RPS + k * ZR, ZR), :])
        return _

    lax.fori_loop(0, RPS // ZR, zero_chunk, None)

    idx_src_cp.wait()
    idx_dst_cp.wait()

    plsc.subcore_barrier()

    gbufs = (gbuf0, gbuf1, gbuf2, gbuf3, gbuf4, gbuf5)

    def task_refs(u, k):
        b = (NRING // 2) * k + (u >> 1)
        if u & 1 == 0:
            return dst_v.at[b], src_v.at[b]
        return src_v.at[b], dst_v.at[b]

    def issue_gather(u, k):
        g_idx, _ = task_refs(u, k)
        pltpu.async_copy(tbl_hbm.at[g_idx], gbufs[u], gsems.at[u])

    def wait_gather(u, k):
        g_idx, _ = task_refs(u, k)
        pltpu.make_async_copy(tbl_hbm.at[g_idx], gbufs[u], gsems.at[u]).wait()

    def issue_scatter(u, k):
        _, s_idx = task_refs(u, k)
        pltpu.async_copy(gbufs[u], acc_s.at[s_idx], ssems.at[u], add=True)

    def wait_scatter(u, k):
        _, s_idx = task_refs(u, k)
        pltpu.make_async_copy(gbufs[u], acc_s.at[s_idx], ssems.at[u]).wait()

    for u in range(NRING):
        issue_gather(u, 0)

    def group(k, _):
        for u in range(NRING):
            wait_gather(u, k)
            issue_scatter(u, k)
        for u in range(NRING):
            wait_scatter(u, k)

        @pl.when(k < NG - 1)
        def _issue_next():
            for u in range(NRING):
                issue_gather(u, k + 1)

        @pl.when(k == NG - 1)
        def _issue_tail():
            for u in range(TAIL):
                issue_gather(u, NG)

        return _

    lax.fori_loop(0, NG, group, None)

    for u in range(TAIL):
        wait_gather(u, NG)
        issue_scatter(u, NG)
    for u in range(TAIL):
        wait_scatter(u, NG)

    plsc.subcore_barrier()

    @pl.when(c == 0)
    def _wb0():
        pltpu.sync_copy(acc_s.at[pl.ds(s * RPS, RPS), :],
                        out0_hbm.at[pl.ds(s * RPS, RPS), :])

    @pl.when(c == 1)
    def _wb1():
        pltpu.sync_copy(acc_s.at[pl.ds(s * RPS, RPS), :],
                        out1_hbm.at[pl.ds(s * RPS, RPS), :])


def _sc_aggregate(tbl, idx):
    mesh = plsc.VectorSubcoreMesh(core_axis_name="c", subcore_axis_name="s",
                                  num_cores=NC, num_subcores=NS)
    f = pl.kernel(
        _sc_agg_body,
        out_type=[jax.ShapeDtypeStruct((ROWS, W), jnp.float32),
                  jax.ShapeDtypeStruct((ROWS, W), jnp.float32)],
        mesh=mesh,
        scratch_types=[
            pltpu.VMEM_SHARED((ROWS, W), jnp.float32),
            pltpu.VMEM((NB, B), jnp.int32),
            pltpu.VMEM((NB, B), jnp.int32),
        ] + [pltpu.VMEM((B, W), jnp.float32)] * NRING + [
            pltpu.VMEM((ZR, W), jnp.float32),
            pltpu.SemaphoreType.DMA((NRING,)),
            pltpu.SemaphoreType.DMA((NRING,)),
        ],
        compiler_params=pltpu.CompilerParams(use_tc_tiling_on_sc=False),
    )
    return f(tbl, idx)


def _tc_tbl_body(x_ref, tbl_ref):
    tbl_ref[:, :F] = x_ref[:, F:]
    one_col = (lax.broadcasted_iota(jnp.int32, (RT, W - F), 1) == 0)
    tbl_ref[:, F:] = one_col.astype(jnp.float32)


def _tc_tbl(x):
    return pl.pallas_call(
        _tc_tbl_body,
        grid=(N // RT,),
        in_specs=[pl.BlockSpec((RT, D), lambda i: (i, 0))],
        out_specs=pl.BlockSpec((RT, W), lambda i: (i, 0)),
        out_shape=jax.ShapeDtypeStruct((ROWS, W), jnp.float32),
    )(x)


def _tc_pre_body(x_ref, w1a_ref, b1_ref, pre_ref):
    pre_ref[...] = jnp.dot(x_ref[:, :F], w1a_ref[...],
                           preferred_element_type=jnp.float32,
                           precision=lax.Precision.HIGHEST) + b1_ref[...]


def _tc_pre(x, W1, b1):
    return pl.pallas_call(
        _tc_pre_body,
        grid=(N // RT,),
        in_specs=[
            pl.BlockSpec((RT, D), lambda i: (i, 0)),
            pl.BlockSpec((F, D), lambda i: (0, 0)),
            pl.BlockSpec((1, D), lambda i: (0, 0)),
        ],
        out_specs=pl.BlockSpec((RT, D), lambda i: (i, 0)),
        out_shape=jax.ShapeDtypeStruct((N, D), jnp.float32),
    )(x, W1[:F], b1.reshape(1, D))


def _tc_mlp_body(pre_ref, p0_ref, p1_ref, w1b_ref, w2_ref, b2_ref, o_ref):
    p = p0_ref[...] + p1_ref[...]
    deg = jnp.maximum(p[:, F:F + 1], 1.0)
    agg = p[:, :F] / deg
    h = pre_ref[...] + jnp.dot(agg, w1b_ref[...],
                               preferred_element_type=jnp.float32,
                               precision=lax.Precision.HIGHEST)
    h = jnp.maximum(h, 0.0)
    o_ref[...] = jnp.dot(h, w2_ref[...],
                         preferred_element_type=jnp.float32,
                         precision=lax.Precision.HIGHEST) + b2_ref[...]


def _tc_mlp(pre, p0, p1, W1, W2, b2):
    return pl.pallas_call(
        _tc_mlp_body,
        grid=(N // RM,),
        in_specs=[
            pl.BlockSpec((RM, D), lambda i: (i, 0)),
            pl.BlockSpec((RM, W), lambda i: (i, 0)),
            pl.BlockSpec((RM, W), lambda i: (i, 0)),
            pl.BlockSpec((F, D), lambda i: (0, 0)),
            pl.BlockSpec((D, D), lambda i: (0, 0)),
            pl.BlockSpec((1, D), lambda i: (0, 0)),
        ],
        out_specs=pl.BlockSpec((RM, D), lambda i: (i, 0)),
        out_shape=jax.ShapeDtypeStruct((N, D), jnp.float32),
    )(pre, p0, p1, W1[F:], W2, b2.reshape(1, D))


def kernel(x, edge_index, W1, b1, W2, b2):
    pad_row = N + (jnp.arange(EP - E, dtype=jnp.int32) % (ROWS - N))
    pad = jnp.stack([pad_row, pad_row])
    idx = jnp.concatenate([edge_index, pad], axis=1).reshape(2, NW, NB, B)

    tbl = _tc_tbl(x)
    p0, p1 = _sc_aggregate(tbl, idx)
    pre = _tc_pre(x, W1, b1)
    return _tc_mlp(pre, p0, p1, W1, W2, b2)


# speedup vs baseline: 1.0152x; 1.0152x over previous
import jax
import jax.numpy as jnp
from jax import lax
from jax.experimental import pallas as pl
from jax.experimental.pallas import tpu as pltpu
from jax.experimental.pallas import tpu_sc as plsc

N = 10000
E = 320000
D = 128
F = 64
W = 72
NC = 2
NS = 16
NW = NC * NS
B = 128
TB = E // B
NBM = TB // NW
XW = TB - NBM * NW
NB = NBM + 1
NRING = 6
ROWS = 10112
RPS = ROWS // NS
ZR = 79
RP = 632
RT = 1000
RM = 2000

T = 2 * NBM
NG = T // NRING


def _sc_agg_body(tbl_hbm, idx_hbm, out0_hbm, out1_hbm, acc_s, src_v, dst_v,
                 gbuf0, gbuf1, gbuf2, gbuf3, gbuf4, gbuf5,
                 zbuf, gsems, ssems):
    c = lax.axis_index("c")
    s = lax.axis_index("s")
    w = c * NS + s

    idx_src_cp = pltpu.async_copy(
        idx_hbm.at[0, pl.ds(w * NBM, NBM)], src_v.at[pl.ds(0, NBM)],
        gsems.at[0])
    idx_dst_cp = pltpu.async_copy(
        idx_hbm.at[1, pl.ds(w * NBM, NBM)], dst_v.at[pl.ds(0, NBM)],
        gsems.at[1])

    @pl.when(w < XW)
    def _stage_extra():
        pltpu.sync_copy(idx_hbm.at[0, pl.ds(NW * NBM + w, 1)],
                        src_v.at[pl.ds(NBM, 1)])
        pltpu.sync_copy(idx_hbm.at[1, pl.ds(NW * NBM + w, 1)],
                        dst_v.at[pl.ds(NBM, 1)])

    z16 = jnp.zeros((16,), jnp.float32)

    def zero_row(r, _):
        for cc in range(W // 16):
            zbuf[r, pl.ds(cc * 16, 16)] = z16
        return _

    lax.fori_loop(0, ZR, zero_row, None)

    def zero_chunk(k, _):
        pltpu.sync_copy(zbuf, acc_s.at[pl.ds(s * RPS + k * ZR, ZR), :])
        return _

    lax.fori_loop(0, RPS // ZR, zero_chunk, None)

    idx_src_cp.wait()
    idx_dst_cp.wait()

    plsc.subcore_barrier()

    gbufs = (gbuf0, gbuf1, gbuf2, gbuf3, gbuf4, gbuf5)

    def task_refs(u, k):
        b = (NRING // 2) * k + (u >> 1)
        if u & 1 == 0:
            return dst_v.at[b], src_v.at[b]
        return src_v.at[b], dst_v.at[b]

    def issue_gather(u, k):
        g_idx, _ = task_refs(u, k)
        pltpu.async_copy(tbl_hbm.at[g_idx], gbufs[u], gsems.at[u])

    def wait_gather(u, k):
        g_idx, _ = task_refs(u, k)
        pltpu.make_async_copy(tbl_hbm.at[g_idx], gbufs[u], gsems.at[u]).wait()

    def issue_scatter(u, k):
        _, s_idx = task_refs(u, k)
        pltpu.async_copy(gbufs[u], acc_s.at[s_idx], ssems.at[u], add=True)

    def wait_scatter(u, k):
        _, s_idx = task_refs(u, k)
        pltpu.make_async_copy(gbufs[u], acc_s.at[s_idx], ssems.at[u]).wait()

    for u in range(NRING):
        issue_gather(u, 0)

    def group(k, _):
        for u in range(NRING):
            wait_gather(u, k)
            issue_scatter(u, k)
        for u in range(NRING):
            wait_scatter(u, k)

        @pl.when(k < NG - 1)
        def _issue_next():
            for u in range(NRING):
                issue_gather(u, k + 1)

        return _

    lax.fori_loop(0, NG, group, None)

    @pl.when(w < XW)
    def _extra_batch():
        pltpu.sync_copy(tbl_hbm.at[dst_v.at[NBM]], gbuf0)
        pltpu.sync_copy(gbuf0, acc_s.at[src_v.at[NBM]], add=True)
        pltpu.sync_copy(tbl_hbm.at[src_v.at[NBM]], gbuf1)
        pltpu.sync_copy(gbuf1, acc_s.at[dst_v.at[NBM]], add=True)

    plsc.subcore_barrier()

    @pl.when(c == 0)
    def _wb0():
        pltpu.sync_copy(acc_s.at[pl.ds(s * RPS, RPS), :],
                        out0_hbm.at[pl.ds(s * RPS, RPS), :])

    @pl.when(c == 1)
    def _wb1():
        pltpu.sync_copy(acc_s.at[pl.ds(s * RPS, RPS), :],
                        out1_hbm.at[pl.ds(s * RPS, RPS), :])


def _sc_aggregate(tbl, idx):
    mesh = plsc.VectorSubcoreMesh(core_axis_name="c", subcore_axis_name="s",
                                  num_cores=NC, num_subcores=NS)
    f = pl.kernel(
        _sc_agg_body,
        out_type=[jax.ShapeDtypeStruct((ROWS, W), jnp.float32),
                  jax.ShapeDtypeStruct((ROWS, W), jnp.float32)],
        mesh=mesh,
        scratch_types=[
            pltpu.VMEM_SHARED((ROWS, W), jnp.float32),
            pltpu.VMEM((NB, B), jnp.int32),
            pltpu.VMEM((NB, B), jnp.int32),
        ] + [pltpu.VMEM((B, W), jnp.float32)] * NRING + [
            pltpu.VMEM((ZR, W), jnp.float32),
            pltpu.SemaphoreType.DMA((NRING,)),
            pltpu.SemaphoreType.DMA((NRING,)),
        ],
        compiler_params=pltpu.CompilerParams(use_tc_tiling_on_sc=False),
    )
    return f(tbl, idx)


def _tc_tbl_body(x_ref, tbl_ref):
    tbl_ref[:, :F] = x_ref[:, F:]
    one_col = (lax.broadcasted_iota(jnp.int32, (RT, W - F), 1) == 0)
    tbl_ref[:, F:] = one_col.astype(jnp.float32)


def _tc_tbl(x):
    return pl.pallas_call(
        _tc_tbl_body,
        grid=(N // RT,),
        in_specs=[pl.BlockSpec((RT, D), lambda i: (i, 0))],
        out_specs=pl.BlockSpec((RT, W), lambda i: (i, 0)),
        out_shape=jax.ShapeDtypeStruct((ROWS, W), jnp.float32),
    )(x)


def _tc_pre_body(x_ref, w1a_ref, b1_ref, pre_ref):
    pre_ref[...] = jnp.dot(x_ref[:, :F], w1a_ref[...],
                           preferred_element_type=jnp.float32,
                           precision=lax.Precision.HIGHEST) + b1_ref[...]


def _tc_pre(x, W1, b1):
    return pl.pallas_call(
        _tc_pre_body,
        grid=(N // RT,),
        in_specs=[
            pl.BlockSpec((RT, D), lambda i: (i, 0)),
            pl.BlockSpec((F, D), lambda i: (0, 0)),
            pl.BlockSpec((1, D), lambda i: (0, 0)),
        ],
        out_specs=pl.BlockSpec((RT, D), lambda i: (i, 0)),
        out_shape=jax.ShapeDtypeStruct((N, D), jnp.float32),
    )(x, W1[:F], b1.reshape(1, D))


def _tc_mlp_body(pre_ref, p0_ref, p1_ref, w1b_ref, w2_ref, b2_ref, o_ref):
    p = p0_ref[...] + p1_ref[...]
    deg = jnp.maximum(p[:, F:F + 1], 1.0)
    agg = p[:, :F] / deg
    h = pre_ref[...] + jnp.dot(agg, w1b_ref[...],
                               preferred_element_type=jnp.float32,
                               precision=lax.Precision.HIGHEST)
    h = jnp.maximum(h, 0.0)
    o_ref[...] = jnp.dot(h, w2_ref[...],
                         preferred_element_type=jnp.float32,
                         precision=lax.Precision.HIGHEST) + b2_ref[...]


def _tc_mlp(pre, p0, p1, W1, W2, b2):
    return pl.pallas_call(
        _tc_mlp_body,
        grid=(N // RM,),
        in_specs=[
            pl.BlockSpec((RM, D), lambda i: (i, 0)),
            pl.BlockSpec((RM, W), lambda i: (i, 0)),
            pl.BlockSpec((RM, W), lambda i: (i, 0)),
            pl.BlockSpec((F, D), lambda i: (0, 0)),
            pl.BlockSpec((D, D), lambda i: (0, 0)),
            pl.BlockSpec((1, D), lambda i: (0, 0)),
        ],
        out_specs=pl.BlockSpec((RM, D), lambda i: (i, 0)),
        out_shape=jax.ShapeDtypeStruct((N, D), jnp.float32),
    )(pre, p0, p1, W1[F:], W2, b2.reshape(1, D))


def kernel(x, edge_index, W1, b1, W2, b2):
    idx = edge_index.reshape(2, TB, B)

    tbl = _tc_tbl(x)
    p0, p1 = _sc_aggregate(tbl, idx)
    pre = _tc_pre(x, W1, b1)
    return _tc_mlp(pre, p0, p1, W1, W2, b2)
